# baseline (speedup 1.0000x reference)
import jax
import jax.numpy as jnp
from jax import lax
from jax.experimental import pallas as pl
from jax.experimental.pallas import tpu as pltpu
from jax.experimental.pallas import tpu_sc as plsc

N = 1024
H = 16
V = 32
E = N * N

NC = 2
NS = 16
L = 16
NW = NC * NS
ROWS_W = N // NW
RC = 8
NCHUNK = ROWS_W // RC
CH = RC * N
HP = 8
NPASS = H // HP


def _sc_body(dist_hbm, tab_hbm, out_hbm, idx_v, tab_v, out_v, sem):
    wid = lax.axis_index("s") * NC + lax.axis_index("c")
    row0 = wid * ROWS_W
    base = row0 * N
    pltpu.sync_copy(tab_hbm, tab_v)

    def chunk_body(c, carry):
        r0 = row0 + c * RC
        off = base + c * CH
        pltpu.sync_copy(dist_hbm.at[pl.ds(r0, RC), :], idx_v)

        for p in range(NPASS):
            def grp_body(g, carry2):
                r = g // (N // L)
                s = pl.multiple_of((g % (N // L)) * L, L)
                d = idx_v[r, pl.ds(s, L)] * H
                vals = [plsc.load_gather(tab_v, [d + (p * HP + hh)])
                        for hh in range(HP)]
                fo = pl.multiple_of(g * L, L)
                for hh in range(HP):
                    out_v[hh, pl.ds(fo, L)] = vals[hh]
                return carry2

            lax.fori_loop(0, CH // L, grp_body, 0)
            copies = [
                pltpu.make_async_copy(
                    out_v.at[hh],
                    out_hbm.at[p * HP + hh, pl.ds(off, CH)], sem)
                for hh in range(HP)
            ]
            for cp in copies:
                cp.start()
            for cp in copies:
                cp.wait()
        return carry

    lax.fori_loop(0, NCHUNK, chunk_body, 0)


def kernel(distances, table):
    tab_flat = table.reshape(V * H)
    k = pl.kernel(
        _sc_body,
        out_type=jax.ShapeDtypeStruct((H, E), jnp.float32),
        mesh=plsc.VectorSubcoreMesh(core_axis_name="c", subcore_axis_name="s"),
        compiler_params=pltpu.CompilerParams(needs_layout_passes=False),
        scratch_types=[
            pltpu.VMEM((RC, N), jnp.int32),
            pltpu.VMEM((V * H,), jnp.float32),
            pltpu.VMEM((HP, CH), jnp.float32),
            pltpu.SemaphoreType.DMA,
        ],
    )
    out = k(distances.astype(jnp.int32), tab_flat)
    return out.reshape(H, N, N)
